# Initial kernel scaffold; baseline (speedup 1.0000x reference)
import jax
import jax.numpy as jnp
from jax.experimental import pallas as pl


def kernel(atomic_numbers, batch, energy_pred, embed, W1, b1, W2, b2, We, be, Wh1, bh1, Wh2, bh2, Wh3, bh3):
    raise NotImplementedError("write your pallas kernel here")


# speedup vs baseline: 8.2878x; 8.2878x over previous
import functools

import jax
import jax.numpy as jnp
from jax import lax
from jax.experimental import pallas as pl
from jax.experimental.pallas import tpu as pltpu
from jax.experimental.pallas import tpu_sc as plsc

N = 100000
B = 512
NZ = 118
HIST = B * NZ
HISTS = HIST + 16
NC = 2
NS = 16
NW = NC * NS
CHUNK = 3136
NPAD = NW * CHUNK


def _silu(x):
    return x / (1.0 + jnp.exp(-x))


def _hist_body(batch_hbm, z_hbm, parts_hbm, batch_v, z_v, hist_v):
    wid = lax.axis_index("s") * NC + lax.axis_index("c")
    base = wid * CHUNK
    pltpu.sync_copy(batch_hbm.at[pl.ds(base, CHUNK)], batch_v)
    pltpu.sync_copy(z_hbm.at[pl.ds(base, CHUNK)], z_v)

    zeros16 = jnp.zeros((16,), jnp.float32)

    def zero_body(i, carry):
        hist_v[pl.ds(i * 16, 16)] = zeros16
        return carry

    lax.fori_loop(0, HISTS // 16, zero_body, 0)

    ones16 = jnp.ones((16,), jnp.float32)

    def acc_body(i, carry):
        off = i * 16
        b16 = batch_v[pl.ds(off, 16)]
        z16 = z_v[pl.ds(off, 16)]
        flat = b16 * NZ + z16
        plsc.addupdate_scatter(hist_v, [flat], ones16)
        return carry

    lax.fori_loop(0, CHUNK // 16, acc_body, 0)
    pltpu.sync_copy(hist_v.at[pl.ds(0, HIST)], parts_hbm.at[wid])


def _dense_body(parts_ref, embed_ref, w1_ref, b1_ref, w2_ref, b2_ref,
                we_ref, be_ref, ep_ref, wh1a_ref, wh1b_ref, wh1c_ref,
                bh1_ref, wh2_ref, bh2_ref, wh3_ref, bh3_ref,
                err_ref, qm_ref, gf_ref):
    dot = functools.partial(jnp.dot, preferred_element_type=jnp.float32,
                            precision=lax.Precision.HIGHEST)
    ddot = functools.partial(jnp.dot, preferred_element_type=jnp.float32)
    counts = jnp.sum(parts_ref[...], axis=0)
    t = _silu(ddot(embed_ref[...], w1_ref[...]) + b1_ref[...])
    t = _silu(ddot(t, w2_ref[...]) + b2_ref[...])
    gf = dot(counts, t)
    qm = dot(gf, we_ref[...]) + be_ref[...]
    x = (dot(gf, wh1a_ref[...]) + qm * wh1b_ref[...]
         + ep_ref[...] * wh1c_ref[...] + bh1_ref[...])
    x = _silu(x)
    x = _silu(dot(x, wh2_ref[...]) + bh2_ref[...])
    e = dot(x, wh3_ref[...]) + bh3_ref[...]
    err_ref[...] = jnp.maximum(e, 0.0) + jnp.log(1.0 + jnp.exp(-jnp.abs(e)))
    qm_ref[...] = qm
    gf_ref[...] = gf


def kernel(atomic_numbers, batch, energy_pred, embed, W1, b1, W2, b2,
           We, be, Wh1, bh1, Wh2, bh2, Wh3, bh3):
    pad_b = jnp.full((NPAD - N,), B, jnp.int32)
    pad_z = jnp.zeros((NPAD - N,), jnp.int32)
    batch_p = jnp.concatenate([batch.astype(jnp.int32), pad_b])
    z_p = jnp.concatenate([atomic_numbers.astype(jnp.int32), pad_z])

    mesh = plsc.VectorSubcoreMesh(core_axis_name="c", subcore_axis_name="s",
                                  num_cores=NC, num_subcores=NS)
    parts = pl.kernel(
        _hist_body,
        out_type=jax.ShapeDtypeStruct((NW, HIST), jnp.float32),
        mesh=mesh,
        compiler_params=pltpu.CompilerParams(needs_layout_passes=False),
        scratch_types=[
            pltpu.VMEM((CHUNK,), jnp.int32),
            pltpu.VMEM((CHUNK,), jnp.int32),
            pltpu.VMEM((HISTS,), jnp.float32),
        ],
    )(batch_p, z_p)

    parts3 = parts.reshape(NW, B, NZ)
    err, qm, gf = pl.pallas_call(
        _dense_body,
        out_shape=[
            jax.ShapeDtypeStruct((B, 1), jnp.float32),
            jax.ShapeDtypeStruct((B, 1), jnp.float32),
            jax.ShapeDtypeStruct((B, 128), jnp.float32),
        ],
    )(parts3, embed, W1, b1.reshape(1, -1), W2, b2.reshape(1, -1),
      We, be.reshape(1, 1), energy_pred.reshape(B, 1),
      Wh1[:128], Wh1[128:129], Wh1[129:130], bh1.reshape(1, -1),
      Wh2, bh2.reshape(1, -1), Wh3, bh3.reshape(1, 1))

    return err[:, 0], qm[:, 0], gf


# speedup vs baseline: 9.9420x; 1.1996x over previous
import functools

import jax
import jax.numpy as jnp
from jax import lax
from jax.experimental import pallas as pl
from jax.experimental.pallas import tpu as pltpu
from jax.experimental.pallas import tpu_sc as plsc

N = 100000
B = 512
NZ = 118
HIST = B * NZ
NC = 2
NS = 16
NW = NC * NS
CHUNK = 3136
TAIL = N - (NW - 1) * CHUNK
SLICE = HIST // NS
ZU = 8


def _silu(x):
    return x / (1.0 + jnp.exp(-x))


def _hist_body(batch_hbm, z_hbm, parts_hbm, batch_v, z_v, hist_v, tmp_v, shared):
    cid = lax.axis_index("c")
    sid = lax.axis_index("s")
    wid = sid * NC + cid
    base = wid * CHUNK

    pltpu.sync_copy(batch_hbm.at[pl.ds(base, TAIL)], batch_v.at[pl.ds(0, TAIL)])
    pltpu.sync_copy(z_hbm.at[pl.ds(base, TAIL)], z_v.at[pl.ds(0, TAIL)])

    @pl.when(wid < NW - 1)
    def _copy_rest():
        pltpu.sync_copy(batch_hbm.at[pl.ds(base + TAIL, CHUNK - TAIL)],
                        batch_v.at[pl.ds(TAIL, CHUNK - TAIL)])
        pltpu.sync_copy(z_hbm.at[pl.ds(base + TAIL, CHUNK - TAIL)],
                        z_v.at[pl.ds(TAIL, CHUNK - TAIL)])

    zeros16 = jnp.zeros((16,), jnp.float32)

    def zero_body(i, carry):
        off = i * (16 * ZU)
        for u in range(ZU):
            hist_v[pl.ds(off + u * 16, 16)] = zeros16
        return carry

    lax.fori_loop(0, HIST // (16 * ZU), zero_body, 0)

    ones16 = jnp.ones((16,), jnp.float32)

    def acc_body(i, carry):
        off = i * 16
        b16 = batch_v[pl.ds(off, 16)]
        z16 = z_v[pl.ds(off, 16)]
        flat = b16 * NZ + z16
        plsc.addupdate_scatter(hist_v, [flat], ones16)
        return carry

    nvec = jnp.where(wid == NW - 1, TAIL // 16, CHUNK // 16)
    lax.fori_loop(0, nvec, acc_body, 0)

    pltpu.sync_copy(hist_v, shared.at[pl.ds(sid * HIST, HIST)])
    plsc.subcore_barrier()

    stripe = sid * SLICE

    def red_body(t, carry):
        row = (sid + t) & (NS - 1)
        pltpu.sync_copy(shared.at[pl.ds(row * HIST + stripe, SLICE)], tmp_v)
        for j in range(SLICE // 16):
            sl = pl.ds(stripe + j * 16, 16)
            tsl = pl.ds(j * 16, 16)
            hist_v[sl] = hist_v[sl] + tmp_v[tsl]
        return carry

    lax.fori_loop(1, NS, red_body, 0)

    pltpu.sync_copy(hist_v.at[pl.ds(stripe, SLICE)],
                    parts_hbm.at[pl.ds(cid * HIST + stripe, SLICE)])


def _dense_body(parts_ref, embed_ref, w1_ref, b1_ref, w2_ref, b2_ref,
                we_ref, be_ref, ep_ref, wh1a_ref, wh1b_ref, wh1c_ref,
                bh1_ref, wh2_ref, bh2_ref, wh3_ref, bh3_ref,
                err_ref, qm_ref, gf_ref):
    dot = functools.partial(jnp.dot, preferred_element_type=jnp.float32,
                            precision=lax.Precision.HIGHEST)
    ddot = functools.partial(jnp.dot, preferred_element_type=jnp.float32)
    counts = parts_ref[0] + parts_ref[1]
    t = _silu(ddot(embed_ref[...], w1_ref[...]) + b1_ref[...])
    t = _silu(ddot(t, w2_ref[...]) + b2_ref[...])
    gf = dot(counts, t)
    qm = dot(gf, we_ref[...]) + be_ref[...]
    x = (dot(gf, wh1a_ref[...]) + qm * wh1b_ref[...]
         + ep_ref[...] * wh1c_ref[...] + bh1_ref[...])
    x = _silu(x)
    x = _silu(dot(x, wh2_ref[...]) + bh2_ref[...])
    e = dot(x, wh3_ref[...]) + bh3_ref[...]
    err_ref[...] = jnp.maximum(e, 0.0) + jnp.log(1.0 + jnp.exp(-jnp.abs(e)))
    qm_ref[...] = qm
    gf_ref[...] = gf


def kernel(atomic_numbers, batch, energy_pred, embed, W1, b1, W2, b2,
           We, be, Wh1, bh1, Wh2, bh2, Wh3, bh3):
    mesh = plsc.VectorSubcoreMesh(core_axis_name="c", subcore_axis_name="s",
                                  num_cores=NC, num_subcores=NS)
    parts = pl.kernel(
        _hist_body,
        out_type=jax.ShapeDtypeStruct((NC * HIST,), jnp.float32),
        mesh=mesh,
        compiler_params=pltpu.CompilerParams(needs_layout_passes=False),
        scratch_types=[
            pltpu.VMEM((CHUNK,), jnp.int32),
            pltpu.VMEM((CHUNK,), jnp.int32),
            pltpu.VMEM((HIST,), jnp.float32),
            pltpu.VMEM((SLICE,), jnp.float32),
            pltpu.VMEM_SHARED((NS * HIST,), jnp.float32),
        ],
    )(batch.astype(jnp.int32), atomic_numbers.astype(jnp.int32))

    parts3 = parts.reshape(NC, B, NZ)
    err, qm, gf = pl.pallas_call(
        _dense_body,
        out_shape=[
            jax.ShapeDtypeStruct((B, 1), jnp.float32),
            jax.ShapeDtypeStruct((B, 1), jnp.float32),
            jax.ShapeDtypeStruct((B, 128), jnp.float32),
        ],
    )(parts3, embed, W1, b1.reshape(1, -1), W2, b2.reshape(1, -1),
      We, be.reshape(1, 1), energy_pred.reshape(B, 1),
      Wh1[:128], Wh1[128:129], Wh1[129:130], bh1.reshape(1, -1),
      Wh2, bh2.reshape(1, -1), Wh3, bh3.reshape(1, 1))

    return err[:, 0], qm[:, 0], gf


# speedup vs baseline: 16.5078x; 1.6604x over previous
import functools

import jax
import jax.numpy as jnp
from jax import lax
from jax.experimental import pallas as pl
from jax.experimental.pallas import tpu as pltpu
from jax.experimental.pallas import tpu_sc as plsc

N = 100000
B = 512
NZ = 118
ROW = 128
HROWS = B * ROW
HSZ = HROWS + ROW
NC = 2
NS = 16
NW = NC * NS
CHUNK = 3136
TAIL = N - (NW - 1) * CHUNK
STR = HROWS // NS
ZU = 8


def _silu(x):
    return x / (1.0 + jnp.exp(-x))


def _hist_body(batch_hbm, z_hbm, parts_hbm,
               batch_v, z_v, hist_v, tmp_v, bnd_v, shared, sbnd):
    cid = lax.axis_index("c")
    sid = lax.axis_index("s")
    wid = sid * NC + cid
    base = wid * CHUNK

    pltpu.sync_copy(batch_hbm.at[pl.ds(base, TAIL)], batch_v.at[pl.ds(0, TAIL)])
    pltpu.sync_copy(z_hbm.at[pl.ds(base, TAIL)], z_v.at[pl.ds(0, TAIL)])

    @pl.when(wid < NW - 1)
    def _copy_rest():
        pltpu.sync_copy(batch_hbm.at[pl.ds(base + TAIL, CHUNK - TAIL)],
                        batch_v.at[pl.ds(TAIL, CHUNK - TAIL)])
        pltpu.sync_copy(z_hbm.at[pl.ds(base + TAIL, CHUNK - TAIL)],
                        z_v.at[pl.ds(TAIL, CHUNK - TAIL)])

    is_last = (wid == NW - 1).astype(jnp.int32)
    b_lo = jnp.min(batch_v[pl.ds(0, 16)])
    last_off = (CHUNK - 16) - (CHUNK - TAIL) * is_last
    b_hi = jnp.max(batch_v[pl.ds(last_off, 16)])
    wlo = (b_lo >> 5) * STR
    whi = ((b_hi >> 5) + 1) * STR

    dump16 = jnp.full((16,), B, jnp.int32)
    zero16i = jnp.zeros((16,), jnp.int32)

    @pl.when(wid == NW - 1)
    def _fill_tail():
        for k in range((CHUNK - TAIL) // 16):
            batch_v[pl.ds(TAIL + k * 16, 16)] = dump16
            z_v[pl.ds(TAIL + k * 16, 16)] = zero16i

    zeros16 = jnp.zeros((16,), jnp.float32)
    ss = sid * STR

    def zero_stripe(i, carry):
        off = ss + i * (16 * ZU)
        for u in range(ZU):
            hist_v[pl.ds(off + u * 16, 16)] = zeros16
        return carry

    lax.fori_loop(0, STR // (16 * ZU), zero_stripe, 0)

    def zero_window(i, carry):
        off = wlo + i * (16 * ZU)
        for u in range(ZU):
            hist_v[pl.ds(off + u * 16, 16)] = zeros16
        return carry

    lax.fori_loop(0, (whi - wlo) // (16 * ZU), zero_window, 0)

    ones16 = jnp.ones((16,), jnp.float32)

    def acc_body(i, carry):
        off = i * 16
        b16 = batch_v[pl.ds(off, 16)]
        z16 = z_v[pl.ds(off, 16)]
        flat = b16 * ROW + z16
        plsc.addupdate_scatter(hist_v, [flat], ones16)
        return carry

    lax.fori_loop(0, CHUNK // 16, acc_body, 0)

    def pub_body(k, carry):
        off = wlo + k * STR
        pltpu.sync_copy(hist_v.at[pl.ds(off, STR)],
                        shared.at[pl.ds(wid * HROWS + off, STR)])
        return carry

    lax.fori_loop(0, (whi - wlo) // STR, pub_body, 0)

    lanes = lax.iota(jnp.int32, 16)
    bnd_v[...] = jnp.where(lanes < 8, wlo, whi)
    pltpu.sync_copy(bnd_v, sbnd.at[pl.ds(sid * 16, 16)])
    plsc.subcore_barrier()

    def red_body(t, carry):
        row = (sid + t) & (NS - 1)
        pltpu.sync_copy(sbnd.at[pl.ds(row * 16, 16)], bnd_v)
        bv = bnd_v[...]
        lo_t = jnp.min(bv)
        hi_t = jnp.max(bv)

        @pl.when((lo_t <= ss) & (ss < hi_t))
        def _add_row():
            pltpu.sync_copy(shared.at[pl.ds((row * NC + cid) * HROWS + ss, STR)],
                            tmp_v)
            for j in range(STR // (16 * ZU)):
                off = j * (16 * ZU)
                for u in range(ZU):
                    sl = pl.ds(ss + off + u * 16, 16)
                    tl = pl.ds(off + u * 16, 16)
                    hist_v[sl] = hist_v[sl] + tmp_v[tl]

        return carry

    lax.fori_loop(1, NS, red_body, 0)

    pltpu.sync_copy(hist_v.at[pl.ds(ss, STR)],
                    parts_hbm.at[pl.ds(cid * HROWS + ss, STR)])


def _dense_body(parts_ref, embed_ref, w1_ref, b1_ref, w2_ref, b2_ref,
                we_ref, be_ref, ep_ref, wh1_ref, bh1_ref, wh2_ref, bh2_ref,
                wh3_ref, bh3_ref, err_ref, qm_ref, gf_ref):
    ddot = functools.partial(jnp.dot, preferred_element_type=jnp.float32)
    counts = parts_ref[0] + parts_ref[1]
    t = _silu(ddot(embed_ref[...], w1_ref[...]) + b1_ref[...])
    t = _silu(ddot(t, w2_ref[...]) + b2_ref[...])
    gf = ddot(counts, t)
    qm = ddot(gf, we_ref[...]) + be_ref[...]
    qe = jnp.concatenate([qm, ep_ref[...]], axis=1)
    x = (ddot(gf, wh1_ref[pl.ds(0, 128), :])
         + ddot(qe, wh1_ref[pl.ds(128, 2), :]) + bh1_ref[...])
    x = _silu(x)
    x = _silu(ddot(x, wh2_ref[...]) + bh2_ref[...])
    e = ddot(x, wh3_ref[...]) + bh3_ref[...]
    err_ref[...] = jnp.maximum(e, 0.0) + jnp.log(1.0 + jnp.exp(-jnp.abs(e)))
    qm_ref[...] = qm
    gf_ref[...] = gf


def kernel(atomic_numbers, batch, energy_pred, embed, W1, b1, W2, b2,
           We, be, Wh1, bh1, Wh2, bh2, Wh3, bh3):
    mesh = plsc.VectorSubcoreMesh(core_axis_name="c", subcore_axis_name="s",
                                  num_cores=NC, num_subcores=NS)
    parts = pl.kernel(
        _hist_body,
        out_type=jax.ShapeDtypeStruct((NC * HROWS,), jnp.float32),
        mesh=mesh,
        compiler_params=pltpu.CompilerParams(needs_layout_passes=False),
        scratch_types=[
            pltpu.VMEM((CHUNK,), jnp.int32),
            pltpu.VMEM((CHUNK,), jnp.int32),
            pltpu.VMEM((HSZ,), jnp.float32),
            pltpu.VMEM((STR,), jnp.float32),
            pltpu.VMEM((16,), jnp.int32),
            pltpu.HBM((NW * HROWS,), jnp.float32),
            pltpu.VMEM_SHARED((NS * 16,), jnp.int32),
        ],
    )(batch.astype(jnp.int32), atomic_numbers.astype(jnp.int32))

    parts3 = parts.reshape(NC, B, ROW)
    embed_p = jnp.pad(embed, ((0, ROW - NZ), (0, 0)))
    err, qm, gf = pl.pallas_call(
        _dense_body,
        out_shape=[
            jax.ShapeDtypeStruct((B, 1), jnp.float32),
            jax.ShapeDtypeStruct((B, 1), jnp.float32),
            jax.ShapeDtypeStruct((B, 128), jnp.float32),
        ],
    )(parts3, embed_p, W1, b1, W2, b2, We, be,
      energy_pred.reshape(B, 1), Wh1, bh1, Wh2, bh2, Wh3, bh3)

    return err[:, 0], qm[:, 0], gf


# speedup vs baseline: 17.4678x; 1.0582x over previous
import functools

import jax
import jax.numpy as jnp
from jax import lax
from jax.experimental import pallas as pl
from jax.experimental.pallas import tpu as pltpu
from jax.experimental.pallas import tpu_sc as plsc

N = 100000
B = 512
NZ = 118
ROW = 128
HROWS = B * ROW
HSZ = HROWS + ROW
NC = 2
NS = 16
NW = NC * NS
CHUNK = 3136
TAIL = N - (NW - 1) * CHUNK
STR = HROWS // NS
ZU = 8


def _silu(x):
    return x / (1.0 + jnp.exp(-x))


def _hist_body(batch_hbm, z_hbm, parts_hbm,
               batch_v, z_v, hist_v, tmp_v, bnd_v, shared, sbnd,
               sem_b, sem_z):
    cid = lax.axis_index("c")
    sid = lax.axis_index("s")
    wid = sid * NC + cid
    base = wid * CHUNK

    cp_b = pltpu.async_copy(batch_hbm.at[pl.ds(base, TAIL)],
                            batch_v.at[pl.ds(0, TAIL)], sem_b)
    cp_z = pltpu.async_copy(z_hbm.at[pl.ds(base, TAIL)],
                            z_v.at[pl.ds(0, TAIL)], sem_z)

    zeros16 = jnp.zeros((16,), jnp.float32)
    ss = sid * STR

    def zero_stripe(i, carry):
        off = ss + i * (16 * ZU)
        for u in range(ZU):
            hist_v[pl.ds(off + u * 16, 16)] = zeros16
        return carry

    lax.fori_loop(0, STR // (16 * ZU), zero_stripe, 0)

    cp_b.wait()
    cp_z.wait()

    @pl.when(wid < NW - 1)
    def _copy_rest():
        pltpu.sync_copy(batch_hbm.at[pl.ds(base + TAIL, CHUNK - TAIL)],
                        batch_v.at[pl.ds(TAIL, CHUNK - TAIL)])
        pltpu.sync_copy(z_hbm.at[pl.ds(base + TAIL, CHUNK - TAIL)],
                        z_v.at[pl.ds(TAIL, CHUNK - TAIL)])

    is_last = (wid == NW - 1).astype(jnp.int32)
    b_lo = jnp.min(batch_v[pl.ds(0, 16)])
    last_off = (CHUNK - 16) - (CHUNK - TAIL) * is_last
    b_hi = jnp.max(batch_v[pl.ds(last_off, 16)])
    wlo = (b_lo >> 5) * STR
    whi = ((b_hi >> 5) + 1) * STR

    dump16 = jnp.full((16,), B, jnp.int32)
    zero16i = jnp.zeros((16,), jnp.int32)

    @pl.when(wid == NW - 1)
    def _fill_tail():
        for k in range((CHUNK - TAIL) // 16):
            batch_v[pl.ds(TAIL + k * 16, 16)] = dump16
            z_v[pl.ds(TAIL + k * 16, 16)] = zero16i

    def zero_window(i, carry):
        off = wlo + i * (16 * ZU)
        for u in range(ZU):
            hist_v[pl.ds(off + u * 16, 16)] = zeros16
        return carry

    lax.fori_loop(0, (whi - wlo) // (16 * ZU), zero_window, 0)

    ones16 = jnp.ones((16,), jnp.float32)

    def acc_body(i, carry):
        for u in range(2):
            off = i * 32 + u * 16
            b16 = batch_v[pl.ds(off, 16)]
            z16 = z_v[pl.ds(off, 16)]
            flat = b16 * ROW + z16
            plsc.addupdate_scatter(hist_v, [flat], ones16)
        return carry

    lax.fori_loop(0, CHUNK // 32, acc_body, 0)

    def pub_body(k, carry):
        off = wlo + k * STR
        pltpu.sync_copy(hist_v.at[pl.ds(off, STR)],
                        shared.at[pl.ds(wid * HROWS + off, STR)])
        return carry

    lax.fori_loop(0, (whi - wlo) // STR, pub_body, 0)

    lanes = lax.iota(jnp.int32, 16)
    bnd_v[...] = jnp.where(lanes < 8, wlo, whi)
    pltpu.sync_copy(bnd_v, sbnd.at[pl.ds(sid * 16, 16)])
    plsc.subcore_barrier()

    def red_body(t, carry):
        row = (sid + t) & (NS - 1)
        pltpu.sync_copy(sbnd.at[pl.ds(row * 16, 16)], bnd_v)
        bv = bnd_v[...]
        lo_t = jnp.min(bv)
        hi_t = jnp.max(bv)

        @pl.when((lo_t <= ss) & (ss < hi_t))
        def _add_row():
            pltpu.sync_copy(shared.at[pl.ds((row * NC + cid) * HROWS + ss, STR)],
                            tmp_v)
            for j in range(STR // (16 * ZU)):
                off = j * (16 * ZU)
                for u in range(ZU):
                    sl = pl.ds(ss + off + u * 16, 16)
                    tl = pl.ds(off + u * 16, 16)
                    hist_v[sl] = hist_v[sl] + tmp_v[tl]

        return carry

    lax.fori_loop(1, NS, red_body, 0)

    pltpu.sync_copy(hist_v.at[pl.ds(ss, STR)],
                    parts_hbm.at[pl.ds(cid * HROWS + ss, STR)])


def _dense_body(parts_ref, embed_ref, w1_ref, b1_ref, w2_ref, b2_ref,
                we_ref, be_ref, ep_ref, wh1_ref, bh1_ref, wh2_ref, bh2_ref,
                wh3_ref, bh3_ref, eq_ref, gf_ref):
    ddot = functools.partial(jnp.dot, preferred_element_type=jnp.float32)
    counts = parts_ref[0] + parts_ref[1]
    t = _silu(ddot(embed_ref[...], w1_ref[...]) + b1_ref[...])
    t = _silu(ddot(t, w2_ref[...]) + b2_ref[...])
    gf = ddot(counts, t)
    qm = ddot(gf, we_ref[...]) + be_ref[...]
    qe = jnp.concatenate([qm, ep_ref[...]], axis=1)
    x = (ddot(gf, wh1_ref[pl.ds(0, 128), :])
         + ddot(qe, wh1_ref[pl.ds(128, 2), :]) + bh1_ref[...])
    x = _silu(x)
    x = _silu(ddot(x, wh2_ref[...]) + bh2_ref[...])
    e = ddot(x, wh3_ref[...]) + bh3_ref[...]
    s = jnp.maximum(e, 0.0) + jnp.log(1.0 + jnp.exp(-jnp.abs(e)))
    eq_ref[...] = jnp.transpose(jnp.concatenate([s, qm], axis=1))
    gf_ref[...] = gf


def kernel(atomic_numbers, batch, energy_pred, embed, W1, b1, W2, b2,
           We, be, Wh1, bh1, Wh2, bh2, Wh3, bh3):
    mesh = plsc.VectorSubcoreMesh(core_axis_name="c", subcore_axis_name="s",
                                  num_cores=NC, num_subcores=NS)
    parts = pl.kernel(
        _hist_body,
        out_type=jax.ShapeDtypeStruct((NC * HROWS,), jnp.float32),
        mesh=mesh,
        compiler_params=pltpu.CompilerParams(needs_layout_passes=False),
        scratch_types=[
            pltpu.VMEM((CHUNK,), jnp.int32),
            pltpu.VMEM((CHUNK,), jnp.int32),
            pltpu.VMEM((HSZ,), jnp.float32),
            pltpu.VMEM((STR,), jnp.float32),
            pltpu.VMEM((16,), jnp.int32),
            pltpu.HBM((NW * HROWS,), jnp.float32),
            pltpu.VMEM_SHARED((NS * 16,), jnp.int32),
            pltpu.SemaphoreType.DMA,
            pltpu.SemaphoreType.DMA,
        ],
    )(batch.astype(jnp.int32), atomic_numbers.astype(jnp.int32))

    parts3 = parts.reshape(NC, B, ROW)
    embed_p = jnp.pad(embed, ((0, ROW - NZ), (0, 0)))
    eq, gf = pl.pallas_call(
        _dense_body,
        out_shape=[
            jax.ShapeDtypeStruct((2, B), jnp.float32),
            jax.ShapeDtypeStruct((B, 128), jnp.float32),
        ],
    )(parts3, embed_p, W1, b1, W2, b2, We, be,
      energy_pred.reshape(B, 1), Wh1, bh1, Wh2, bh2, Wh3, bh3)

    return eq[0], eq[1], gf


# speedup vs baseline: 18.0607x; 1.0339x over previous
import functools

import jax
import jax.numpy as jnp
from jax import lax
from jax.experimental import pallas as pl
from jax.experimental.pallas import tpu as pltpu
from jax.experimental.pallas import tpu_sc as plsc

N = 100000
B = 512
NZ = 118
ROW = 128
HROWS = B * ROW
HSZ = HROWS + ROW
NC = 2
NS = 16
NW = NC * NS
CHUNK = 3136
TAIL = N - (NW - 1) * CHUNK
STR = HROWS // NS
ZU = 8


def _silu(x):
    return x / (1.0 + jnp.exp(-x))


def _hist_body(batch_hbm, z_hbm, parts_hbm,
               batch_v, z_v, hist_v, tmp_v, bnd_v, bndall_v, shared, sbnd,
               sem_b, sem_z):
    cid = lax.axis_index("c")
    sid = lax.axis_index("s")
    wid = sid * NC + cid
    base = wid * CHUNK

    cp_b = pltpu.async_copy(batch_hbm.at[pl.ds(base, TAIL)],
                            batch_v.at[pl.ds(0, TAIL)], sem_b)
    cp_z = pltpu.async_copy(z_hbm.at[pl.ds(base, TAIL)],
                            z_v.at[pl.ds(0, TAIL)], sem_z)

    zeros16 = jnp.zeros((16,), jnp.float32)
    ss = sid * STR

    def zero_stripe(i, carry):
        off = ss + i * (16 * ZU)
        for u in range(ZU):
            hist_v[pl.ds(off + u * 16, 16)] = zeros16
        return carry

    lax.fori_loop(0, STR // (16 * ZU), zero_stripe, 0)

    cp_b.wait()
    cp_z.wait()

    @pl.when(wid < NW - 1)
    def _copy_rest():
        pltpu.sync_copy(batch_hbm.at[pl.ds(base + TAIL, CHUNK - TAIL)],
                        batch_v.at[pl.ds(TAIL, CHUNK - TAIL)])
        pltpu.sync_copy(z_hbm.at[pl.ds(base + TAIL, CHUNK - TAIL)],
                        z_v.at[pl.ds(TAIL, CHUNK - TAIL)])

    is_last = (wid == NW - 1).astype(jnp.int32)
    b_lo = jnp.min(batch_v[pl.ds(0, 16)])
    last_off = (CHUNK - 16) - (CHUNK - TAIL) * is_last
    b_hi = jnp.max(batch_v[pl.ds(last_off, 16)])
    wlo = (b_lo >> 5) * STR
    whi = ((b_hi >> 5) + 1) * STR

    lanes = lax.iota(jnp.int32, 16)
    bnd_v[...] = jnp.where(lanes < 8, wlo, whi)
    cp_bnd = pltpu.async_copy(bnd_v, sbnd.at[pl.ds(sid * 16, 16)], sem_b)

    dump16 = jnp.full((16,), B, jnp.int32)
    zero16i = jnp.zeros((16,), jnp.int32)

    @pl.when(wid == NW - 1)
    def _fill_tail():
        for k in range((CHUNK - TAIL) // 16):
            batch_v[pl.ds(TAIL + k * 16, 16)] = dump16
            z_v[pl.ds(TAIL + k * 16, 16)] = zero16i

    def zero_window(i, carry):
        off = wlo + i * (16 * ZU)
        for u in range(ZU):
            hist_v[pl.ds(off + u * 16, 16)] = zeros16
        return carry

    lax.fori_loop(0, (whi - wlo) // (16 * ZU), zero_window, 0)

    ones16 = jnp.ones((16,), jnp.float32)

    def acc_body(i, carry):
        for u in range(2):
            off = i * 32 + u * 16
            b16 = batch_v[pl.ds(off, 16)]
            z16 = z_v[pl.ds(off, 16)]
            flat = b16 * ROW + z16
            plsc.addupdate_scatter(hist_v, [flat], ones16)
        return carry

    lax.fori_loop(0, CHUNK // 32, acc_body, 0)

    def pub_body(k, carry):
        off = wlo + k * STR
        pltpu.async_copy(hist_v.at[pl.ds(off, STR)],
                         shared.at[pl.ds(wid * HROWS + off, STR)], sem_z)
        return carry

    nstr = (whi - wlo) // STR
    lax.fori_loop(0, nstr, pub_body, 0)

    def pub_drain(k, carry):
        off = wlo + k * STR
        pltpu.make_async_copy(hist_v.at[pl.ds(off, STR)],
                              shared.at[pl.ds(wid * HROWS + off, STR)],
                              sem_z).wait()
        return carry

    lax.fori_loop(0, nstr, pub_drain, 0)
    cp_bnd.wait()
    plsc.subcore_barrier()

    pltpu.sync_copy(sbnd, bndall_v)

    def red_body(t, carry):
        row = (sid + t) & (NS - 1)
        bv = bndall_v[pl.ds(row * 16, 16)]
        lo_t = jnp.min(bv)
        hi_t = jnp.max(bv)

        @pl.when((lo_t <= ss) & (ss < hi_t))
        def _add_row():
            pltpu.sync_copy(shared.at[pl.ds((row * NC + cid) * HROWS + ss, STR)],
                            tmp_v)
            for j in range(STR // (16 * ZU)):
                off = j * (16 * ZU)
                for u in range(ZU):
                    sl = pl.ds(ss + off + u * 16, 16)
                    tl = pl.ds(off + u * 16, 16)
                    hist_v[sl] = hist_v[sl] + tmp_v[tl]

        return carry

    lax.fori_loop(1, NS, red_body, 0)

    pltpu.sync_copy(hist_v.at[pl.ds(ss, STR)],
                    parts_hbm.at[pl.ds(cid * HROWS + ss, STR)])


def _dense_body(parts_ref, embed_ref, w1_ref, b1_ref, w2_ref, b2_ref,
                we_ref, be_ref, ep_ref, wh1_ref, bh1_ref, wh2_ref, bh2_ref,
                wh3_ref, bh3_ref, eq_ref, gf_ref):
    ddot = functools.partial(jnp.dot, preferred_element_type=jnp.float32)
    counts = parts_ref[0] + parts_ref[1]
    t = _silu(ddot(embed_ref[...], w1_ref[...]) + b1_ref[...])
    t = _silu(ddot(t, w2_ref[...]) + b2_ref[...])
    gf = ddot(counts, t)
    qm = ddot(gf, we_ref[...]) + be_ref[...]
    qe = jnp.concatenate([qm, ep_ref[...]], axis=1)
    x = (ddot(gf, wh1_ref[pl.ds(0, 128), :])
         + ddot(qe, wh1_ref[pl.ds(128, 2), :]) + bh1_ref[...])
    x = _silu(x)
    x = _silu(ddot(x, wh2_ref[...]) + bh2_ref[...])
    e = ddot(x, wh3_ref[...]) + bh3_ref[...]
    s = jnp.maximum(e, 0.0) + jnp.log(1.0 + jnp.exp(-jnp.abs(e)))
    eq_ref[...] = jnp.transpose(jnp.concatenate([s, qm], axis=1))
    gf_ref[...] = gf


def kernel(atomic_numbers, batch, energy_pred, embed, W1, b1, W2, b2,
           We, be, Wh1, bh1, Wh2, bh2, Wh3, bh3):
    mesh = plsc.VectorSubcoreMesh(core_axis_name="c", subcore_axis_name="s",
                                  num_cores=NC, num_subcores=NS)
    parts = pl.kernel(
        _hist_body,
        out_type=jax.ShapeDtypeStruct((NC * HROWS,), jnp.float32),
        mesh=mesh,
        compiler_params=pltpu.CompilerParams(needs_layout_passes=False),
        scratch_types=[
            pltpu.VMEM((CHUNK,), jnp.int32),
            pltpu.VMEM((CHUNK,), jnp.int32),
            pltpu.VMEM((HSZ,), jnp.float32),
            pltpu.VMEM((STR,), jnp.float32),
            pltpu.VMEM((16,), jnp.int32),
            pltpu.VMEM((NS * 16,), jnp.int32),
            pltpu.HBM((NW * HROWS,), jnp.float32),
            pltpu.VMEM_SHARED((NS * 16,), jnp.int32),
            pltpu.SemaphoreType.DMA,
            pltpu.SemaphoreType.DMA,
        ],
    )(batch.astype(jnp.int32), atomic_numbers.astype(jnp.int32))

    parts3 = parts.reshape(NC, B, ROW)
    embed_p = jnp.pad(embed, ((0, ROW - NZ), (0, 0)))
    eq, gf = pl.pallas_call(
        _dense_body,
        out_shape=[
            jax.ShapeDtypeStruct((2, B), jnp.float32),
            jax.ShapeDtypeStruct((B, 128), jnp.float32),
        ],
    )(parts3, embed_p, W1, b1, W2, b2, We, be,
      energy_pred.reshape(B, 1), Wh1, bh1, Wh2, bh2, Wh3, bh3)

    return eq[0], eq[1], gf


# speedup vs baseline: 18.0765x; 1.0009x over previous
import functools

import jax
import jax.numpy as jnp
from jax import lax
from jax.experimental import pallas as pl
from jax.experimental.pallas import tpu as pltpu
from jax.experimental.pallas import tpu_sc as plsc

N = 100000
B = 512
NZ = 118
ROW = 128
HROWS = B * ROW
HSZ = HROWS + ROW
NC = 2
NS = 16
NW = NC * NS
CHUNK = 3136
TAIL = N - (NW - 1) * CHUNK
STR = HROWS // NS
ZU = 8


def _silu(x):
    return x / (1.0 + jnp.exp(-x))


def _hist_body(batch_hbm, z_hbm, parts_hbm,
               batch_v, z_v, hist_v, tmp_v, bnd_v, bndall_v, shared, sbnd,
               sem_b, sem_z):
    cid = lax.axis_index("c")
    sid = lax.axis_index("s")
    wid = sid * NC + cid
    base = wid * CHUNK

    cp_b = pltpu.async_copy(batch_hbm.at[pl.ds(base, TAIL)],
                            batch_v.at[pl.ds(0, TAIL)], sem_b)
    cp_z = pltpu.async_copy(z_hbm.at[pl.ds(base, TAIL)],
                            z_v.at[pl.ds(0, TAIL)], sem_z)

    zeros16 = jnp.zeros((16,), jnp.float32)
    ss = sid * STR

    def zero_stripe(i, carry):
        off = ss + i * (16 * ZU)
        for u in range(ZU):
            hist_v[pl.ds(off + u * 16, 16)] = zeros16
        return carry

    lax.fori_loop(0, STR // (16 * ZU), zero_stripe, 0)

    cp_b.wait()
    cp_z.wait()

    @pl.when(wid < NW - 1)
    def _copy_rest():
        pltpu.sync_copy(batch_hbm.at[pl.ds(base + TAIL, CHUNK - TAIL)],
                        batch_v.at[pl.ds(TAIL, CHUNK - TAIL)])
        pltpu.sync_copy(z_hbm.at[pl.ds(base + TAIL, CHUNK - TAIL)],
                        z_v.at[pl.ds(TAIL, CHUNK - TAIL)])

    is_last = (wid == NW - 1).astype(jnp.int32)
    b_lo = jnp.min(batch_v[pl.ds(0, 16)])
    last_off = (CHUNK - 16) - (CHUNK - TAIL) * is_last
    b_hi = jnp.max(batch_v[pl.ds(last_off, 16)])
    wlo = (b_lo >> 5) * STR
    whi = ((b_hi >> 5) + 1) * STR

    lanes = lax.iota(jnp.int32, 16)
    bnd_v[...] = jnp.where(lanes < 8, wlo, whi)
    cp_bnd = pltpu.async_copy(bnd_v, sbnd.at[pl.ds(sid * 16, 16)], sem_b)

    dump16 = jnp.full((16,), B, jnp.int32)
    zero16i = jnp.zeros((16,), jnp.int32)

    @pl.when(wid == NW - 1)
    def _fill_tail():
        for k in range((CHUNK - TAIL) // 16):
            batch_v[pl.ds(TAIL + k * 16, 16)] = dump16
            z_v[pl.ds(TAIL + k * 16, 16)] = zero16i

    def zero_window(i, carry):
        off = wlo + i * (16 * ZU)
        for u in range(ZU):
            hist_v[pl.ds(off + u * 16, 16)] = zeros16
        return carry

    lax.fori_loop(0, (whi - wlo) // (16 * ZU), zero_window, 0)

    ones16 = jnp.ones((16,), jnp.float32)

    def acc_body(i, carry):
        for u in range(2):
            off = i * 32 + u * 16
            b16 = batch_v[pl.ds(off, 16)]
            z16 = z_v[pl.ds(off, 16)]
            flat = b16 * ROW + z16
            plsc.addupdate_scatter(hist_v, [flat], ones16)
        return carry

    lax.fori_loop(0, CHUNK // 32, acc_body, 0)

    def pub_body(k, carry):
        off = wlo + k * STR
        pltpu.async_copy(hist_v.at[pl.ds(off, STR)],
                         shared.at[pl.ds(wid * HROWS + off, STR)], sem_z)
        return carry

    nstr = (whi - wlo) // STR
    lax.fori_loop(0, nstr, pub_body, 0)

    def pub_drain(k, carry):
        off = wlo + k * STR
        pltpu.make_async_copy(hist_v.at[pl.ds(off, STR)],
                              shared.at[pl.ds(wid * HROWS + off, STR)],
                              sem_z).wait()
        return carry

    lax.fori_loop(0, nstr, pub_drain, 0)
    cp_bnd.wait()
    plsc.subcore_barrier()

    pltpu.sync_copy(sbnd, bndall_v)

    def red_body(t, carry):
        row = (sid + t) & (NS - 1)
        bv = bndall_v[pl.ds(row * 16, 16)]
        lo_t = jnp.min(bv)
        hi_t = jnp.max(bv)

        @pl.when((lo_t <= ss) & (ss < hi_t))
        def _add_row():
            pltpu.sync_copy(shared.at[pl.ds((row * NC + cid) * HROWS + ss, STR)],
                            tmp_v)
            for j in range(STR // (16 * ZU)):
                off = j * (16 * ZU)
                for u in range(ZU):
                    sl = pl.ds(ss + off + u * 16, 16)
                    tl = pl.ds(off + u * 16, 16)
                    hist_v[sl] = hist_v[sl] + tmp_v[tl]

        return carry

    lax.fori_loop(1, NS, red_body, 0)

    pltpu.sync_copy(hist_v.at[pl.ds(ss, STR)],
                    parts_hbm.at[pl.ds(cid * HROWS + ss, STR)])


def _dense_body(parts_ref, embed_ref, w1_ref, b1_ref, w2_ref, b2_ref,
                we_ref, be_ref, ep_ref, wh1_ref, bh1_ref, wh2_ref, bh2_ref,
                wh3_ref, bh3_ref, eq_ref, gf_ref):
    ddot = functools.partial(jnp.dot, preferred_element_type=jnp.float32)
    counts = jnp.reshape(parts_ref[pl.ds(0, HROWS)]
                         + parts_ref[pl.ds(HROWS, HROWS)], (B, ROW))
    t = _silu(ddot(embed_ref[...], w1_ref[...]) + b1_ref[...])
    t = _silu(ddot(t, w2_ref[...]) + b2_ref[...])
    gf = ddot(counts, t)
    qm = ddot(gf, we_ref[...]) + be_ref[...]
    qe = jnp.concatenate([qm, ep_ref[...]], axis=1)
    x = (ddot(gf, wh1_ref[pl.ds(0, 128), :])
         + ddot(qe, wh1_ref[pl.ds(128, 2), :]) + bh1_ref[...])
    x = _silu(x)
    x = _silu(ddot(x, wh2_ref[...]) + bh2_ref[...])
    e = ddot(x, wh3_ref[...]) + bh3_ref[...]
    s = jnp.maximum(e, 0.0) + jnp.log(1.0 + jnp.exp(-jnp.abs(e)))
    eq_ref[...] = jnp.transpose(jnp.concatenate([s, qm], axis=1))
    gf_ref[...] = gf


def kernel(atomic_numbers, batch, energy_pred, embed, W1, b1, W2, b2,
           We, be, Wh1, bh1, Wh2, bh2, Wh3, bh3):
    mesh = plsc.VectorSubcoreMesh(core_axis_name="c", subcore_axis_name="s",
                                  num_cores=NC, num_subcores=NS)
    parts = pl.kernel(
        _hist_body,
        out_type=jax.ShapeDtypeStruct((NC * HROWS,), jnp.float32),
        mesh=mesh,
        compiler_params=pltpu.CompilerParams(needs_layout_passes=False),
        scratch_types=[
            pltpu.VMEM((CHUNK,), jnp.int32),
            pltpu.VMEM((CHUNK,), jnp.int32),
            pltpu.VMEM((HSZ,), jnp.float32),
            pltpu.VMEM((STR,), jnp.float32),
            pltpu.VMEM((16,), jnp.int32),
            pltpu.VMEM((NS * 16,), jnp.int32),
            pltpu.HBM((NW * HROWS,), jnp.float32),
            pltpu.VMEM_SHARED((NS * 16,), jnp.int32),
            pltpu.SemaphoreType.DMA,
            pltpu.SemaphoreType.DMA,
        ],
    )(batch.astype(jnp.int32), atomic_numbers.astype(jnp.int32))

    embed_p = jnp.pad(embed, ((0, ROW - NZ), (0, 0)))
    eq, gf = pl.pallas_call(
        _dense_body,
        out_shape=[
            jax.ShapeDtypeStruct((2, B), jnp.float32),
            jax.ShapeDtypeStruct((B, 128), jnp.float32),
        ],
    )(parts, embed_p, W1, b1, W2, b2, We, be,
      energy_pred.reshape(B, 1), Wh1, bh1, Wh2, bh2, Wh3, bh3)

    return eq[0], eq[1], gf


# speedup vs baseline: 18.7917x; 1.0396x over previous
import functools

import jax
import jax.numpy as jnp
from jax import lax
from jax.experimental import pallas as pl
from jax.experimental.pallas import tpu as pltpu
from jax.experimental.pallas import tpu_sc as plsc

N = 100000
B = 512
NZ = 118
ROW = 128
HROWS = B * ROW
HSZ = HROWS + ROW
NC = 2
NS = 16
NW = NC * NS
CHUNK = 3136
TAIL = N - (NW - 1) * CHUNK
STR = HROWS // NS
ZU = 8


def _silu(x):
    return x / (1.0 + jnp.exp(-x))


def _hist_body(batch_hbm, z_hbm, parts_hbm,
               batch_v, z_v, hist_v, tmp_v, bnd_v, bndall_v, shared, sbnd,
               sem_b, sem_z):
    cid = lax.axis_index("c")
    sid = lax.axis_index("s")
    wid = sid * NC + cid
    base = wid * CHUNK

    cp_b = pltpu.async_copy(batch_hbm.at[pl.ds(base, TAIL)],
                            batch_v.at[pl.ds(0, TAIL)], sem_b)
    cp_z = pltpu.async_copy(z_hbm.at[pl.ds(base, TAIL)],
                            z_v.at[pl.ds(0, TAIL)], sem_z)

    zeros16 = jnp.zeros((16,), jnp.float32)
    ss = sid * STR

    def zero_stripe(i, carry):
        off = ss + i * (16 * ZU)
        for u in range(ZU):
            hist_v[pl.ds(off + u * 16, 16)] = zeros16
        return carry

    lax.fori_loop(0, STR // (16 * ZU), zero_stripe, 0)

    cp_b.wait()
    cp_z.wait()

    @pl.when(wid < NW - 1)
    def _copy_rest():
        pltpu.sync_copy(batch_hbm.at[pl.ds(base + TAIL, CHUNK - TAIL)],
                        batch_v.at[pl.ds(TAIL, CHUNK - TAIL)])
        pltpu.sync_copy(z_hbm.at[pl.ds(base + TAIL, CHUNK - TAIL)],
                        z_v.at[pl.ds(TAIL, CHUNK - TAIL)])

    is_last = (wid == NW - 1).astype(jnp.int32)
    b_lo = jnp.min(batch_v[pl.ds(0, 16)])
    last_off = (CHUNK - 16) - (CHUNK - TAIL) * is_last
    b_hi = jnp.max(batch_v[pl.ds(last_off, 16)])
    wlo = (b_lo >> 5) * STR
    whi = ((b_hi >> 5) + 1) * STR

    lanes = lax.iota(jnp.int32, 16)
    bnd_v[...] = jnp.where(lanes < 8, wlo, whi)
    cp_bnd = pltpu.async_copy(bnd_v, sbnd.at[pl.ds(sid * 16, 16)], sem_b)

    dump16 = jnp.full((16,), B, jnp.int32)
    zero16i = jnp.zeros((16,), jnp.int32)

    @pl.when(wid == NW - 1)
    def _fill_tail():
        for k in range((CHUNK - TAIL) // 16):
            batch_v[pl.ds(TAIL + k * 16, 16)] = dump16
            z_v[pl.ds(TAIL + k * 16, 16)] = zero16i

    def zero_window(i, carry):
        off = wlo + i * (16 * ZU)
        for u in range(ZU):
            hist_v[pl.ds(off + u * 16, 16)] = zeros16
        return carry

    lax.fori_loop(0, (whi - wlo) // (16 * ZU), zero_window, 0)

    ones16 = jnp.ones((16,), jnp.float32)

    def acc_body(i, carry):
        for u in range(4):
            off = i * 64 + u * 16
            b16 = batch_v[pl.ds(off, 16)]
            z16 = z_v[pl.ds(off, 16)]
            flat = b16 * ROW + z16
            plsc.addupdate_scatter(hist_v, [flat], ones16)
        return carry

    lax.fori_loop(0, CHUNK // 64, acc_body, 0)

    def pub_body(k, carry):
        off = wlo + k * STR
        pltpu.async_copy(hist_v.at[pl.ds(off, STR)],
                         shared.at[pl.ds(wid * HROWS + off, STR)], sem_z)
        return carry

    nstr = (whi - wlo) // STR
    lax.fori_loop(0, nstr, pub_body, 0)

    def pub_drain(k, carry):
        off = wlo + k * STR
        pltpu.make_async_copy(hist_v.at[pl.ds(off, STR)],
                              shared.at[pl.ds(wid * HROWS + off, STR)],
                              sem_z).wait()
        return carry

    lax.fori_loop(0, nstr, pub_drain, 0)
    cp_bnd.wait()
    plsc.subcore_barrier()

    pltpu.sync_copy(sbnd, bndall_v)

    def red_body(t, carry):
        row = (sid + t) & (NS - 1)
        bv = bndall_v[pl.ds(row * 16, 16)]
        lo_t = jnp.min(bv)
        hi_t = jnp.max(bv)

        @pl.when((lo_t <= ss) & (ss < hi_t))
        def _add_row():
            pltpu.sync_copy(shared.at[pl.ds((row * NC + cid) * HROWS + ss, STR)],
                            tmp_v)
            for j in range(STR // (16 * ZU)):
                off = j * (16 * ZU)
                for u in range(ZU):
                    sl = pl.ds(ss + off + u * 16, 16)
                    tl = pl.ds(off + u * 16, 16)
                    hist_v[sl] = hist_v[sl] + tmp_v[tl]

        return carry

    lax.fori_loop(1, NS, red_body, 0)

    pltpu.sync_copy(hist_v.at[pl.ds(ss, STR)],
                    parts_hbm.at[pl.ds(cid * HROWS + ss, STR)])


def _dense_body(parts_ref, embed_ref, w1_ref, b1_ref, w2_ref, b2_ref,
                we_ref, be_ref, ep_ref, wh1_ref, bh1_ref, wh2_ref, bh2_ref,
                wh3_ref, bh3_ref, err_ref, qm_ref, gf_ref):
    ddot = functools.partial(jnp.dot, preferred_element_type=jnp.float32)
    counts = jnp.reshape(parts_ref[pl.ds(0, HROWS)]
                         + parts_ref[pl.ds(HROWS, HROWS)], (B, ROW))
    t = _silu(ddot(embed_ref[...], w1_ref[...]) + b1_ref[...])
    t = _silu(ddot(t, w2_ref[...]) + b2_ref[...])
    gf = ddot(counts, t)
    qm = ddot(gf, we_ref[...]) + be_ref[...]
    qe = jnp.concatenate([qm, ep_ref[...]], axis=1)
    x = (ddot(gf, wh1_ref[pl.ds(0, 128), :])
         + ddot(qe, wh1_ref[pl.ds(128, 2), :]) + bh1_ref[...])
    x = _silu(x)
    x = _silu(ddot(x, wh2_ref[...]) + bh2_ref[...])
    e = ddot(x, wh3_ref[...]) + bh3_ref[...]
    s = jnp.maximum(e, 0.0) + jnp.log(1.0 + jnp.exp(-jnp.abs(e)))
    eq = jnp.transpose(jnp.concatenate([s, qm], axis=1))
    err_ref[...] = eq[0]
    qm_ref[...] = eq[1]
    gf_ref[...] = gf


def kernel(atomic_numbers, batch, energy_pred, embed, W1, b1, W2, b2,
           We, be, Wh1, bh1, Wh2, bh2, Wh3, bh3):
    mesh = plsc.VectorSubcoreMesh(core_axis_name="c", subcore_axis_name="s",
                                  num_cores=NC, num_subcores=NS)
    parts = pl.kernel(
        _hist_body,
        out_type=jax.ShapeDtypeStruct((NC * HROWS,), jnp.float32),
        mesh=mesh,
        compiler_params=pltpu.CompilerParams(needs_layout_passes=False),
        scratch_types=[
            pltpu.VMEM((CHUNK,), jnp.int32),
            pltpu.VMEM((CHUNK,), jnp.int32),
            pltpu.VMEM((HSZ,), jnp.float32),
            pltpu.VMEM((STR,), jnp.float32),
            pltpu.VMEM((16,), jnp.int32),
            pltpu.VMEM((NS * 16,), jnp.int32),
            pltpu.HBM((NW * HROWS,), jnp.float32),
            pltpu.VMEM_SHARED((NS * 16,), jnp.int32),
            pltpu.SemaphoreType.DMA,
            pltpu.SemaphoreType.DMA,
        ],
    )(batch.astype(jnp.int32), atomic_numbers.astype(jnp.int32))

    embed_p = jnp.pad(embed, ((0, ROW - NZ), (0, 0)))
    err, qm, gf = pl.pallas_call(
        _dense_body,
        out_shape=[
            jax.ShapeDtypeStruct((B,), jnp.float32),
            jax.ShapeDtypeStruct((B,), jnp.float32),
            jax.ShapeDtypeStruct((B, 128), jnp.float32),
        ],
    )(parts, embed_p, W1, b1, W2, b2, We, be,
      energy_pred.reshape(B, 1), Wh1, bh1, Wh2, bh2, Wh3, bh3)

    return err, qm, gf


# speedup vs baseline: 19.0942x; 1.0161x over previous
import functools

import jax
import jax.numpy as jnp
from jax import lax
from jax.experimental import pallas as pl
from jax.experimental.pallas import tpu as pltpu
from jax.experimental.pallas import tpu_sc as plsc

N = 100000
B = 512
NZ = 118
ROW = 128
HROWS = B * ROW
HSZ = HROWS + ROW
NC = 2
NS = 16
NW = NC * NS
CHUNK = 3136
TAIL = N - (NW - 1) * CHUNK
STR = HROWS // NS
WCAP = 4
ZU = 8


def _silu(x):
    return x / (1.0 + jnp.exp(-x))


def _hist_body(batch_hbm, z_hbm, parts_hbm,
               batch_v, z_v, hist_v, tmp_v, bnd_v, bndall_v, shared, sstage,
               sbnd, sem_b, sem_z):
    cid = lax.axis_index("c")
    sid = lax.axis_index("s")
    wid = sid * NC + cid
    base = wid * CHUNK

    cp_b = pltpu.async_copy(batch_hbm.at[pl.ds(base, TAIL)],
                            batch_v.at[pl.ds(0, TAIL)], sem_b)
    cp_z = pltpu.async_copy(z_hbm.at[pl.ds(base, TAIL)],
                            z_v.at[pl.ds(0, TAIL)], sem_z)

    zeros16 = jnp.zeros((16,), jnp.float32)
    ss = sid * STR

    def zero_stripe(i, carry):
        off = ss + i * (16 * ZU)
        for u in range(ZU):
            hist_v[pl.ds(off + u * 16, 16)] = zeros16
        return carry

    lax.fori_loop(0, STR // (16 * ZU), zero_stripe, 0)

    cp_b.wait()
    cp_z.wait()

    @pl.when(wid < NW - 1)
    def _copy_rest():
        pltpu.sync_copy(batch_hbm.at[pl.ds(base + TAIL, CHUNK - TAIL)],
                        batch_v.at[pl.ds(TAIL, CHUNK - TAIL)])
        pltpu.sync_copy(z_hbm.at[pl.ds(base + TAIL, CHUNK - TAIL)],
                        z_v.at[pl.ds(TAIL, CHUNK - TAIL)])

    is_last = (wid == NW - 1).astype(jnp.int32)
    b_lo = jnp.min(batch_v[pl.ds(0, 16)])
    last_off = (CHUNK - 16) - (CHUNK - TAIL) * is_last
    b_hi = jnp.max(batch_v[pl.ds(last_off, 16)])
    wlo = (b_lo >> 5) * STR
    whi = ((b_hi >> 5) + 1) * STR

    lanes = lax.iota(jnp.int32, 16)
    bnd_v[...] = jnp.where(lanes < 8, wlo, whi)
    cp_bnd = pltpu.async_copy(bnd_v, sbnd.at[pl.ds(sid * 16, 16)], sem_b)

    dump16 = jnp.full((16,), B, jnp.int32)
    zero16i = jnp.zeros((16,), jnp.int32)

    @pl.when(wid == NW - 1)
    def _fill_tail():
        for k in range((CHUNK - TAIL) // 16):
            batch_v[pl.ds(TAIL + k * 16, 16)] = dump16
            z_v[pl.ds(TAIL + k * 16, 16)] = zero16i

    def zero_window(i, carry):
        off = wlo + i * (16 * ZU)
        for u in range(ZU):
            hist_v[pl.ds(off + u * 16, 16)] = zeros16
        return carry

    lax.fori_loop(0, (whi - wlo) // (16 * ZU), zero_window, 0)

    ones16 = jnp.ones((16,), jnp.float32)

    def acc_body(i, carry):
        for u in range(4):
            off = i * 64 + u * 16
            b16 = batch_v[pl.ds(off, 16)]
            z16 = z_v[pl.ds(off, 16)]
            flat = b16 * ROW + z16
            plsc.addupdate_scatter(hist_v, [flat], ones16)
        return carry

    lax.fori_loop(0, CHUNK // 64, acc_body, 0)

    def pub_body(k, carry):
        off = wlo + k * STR

        @pl.when(k < WCAP)
        def _to_spmem():
            pltpu.async_copy(hist_v.at[pl.ds(off, STR)],
                             sstage.at[pl.ds((sid * WCAP + k) * STR, STR)],
                             sem_z)

        @pl.when(k >= WCAP)
        def _to_hbm():
            pltpu.async_copy(hist_v.at[pl.ds(off, STR)],
                             shared.at[pl.ds(wid * HROWS + off, STR)], sem_z)

        return carry

    nstr = (whi - wlo) // STR
    lax.fori_loop(0, nstr, pub_body, 0)

    def pub_drain(k, carry):
        off = wlo + k * STR
        pltpu.make_async_copy(hist_v.at[pl.ds(off, STR)],
                              shared.at[pl.ds(wid * HROWS + off, STR)],
                              sem_z).wait()
        return carry

    lax.fori_loop(0, nstr, pub_drain, 0)
    cp_bnd.wait()
    plsc.subcore_barrier()

    pltpu.sync_copy(sbnd, bndall_v)

    def red_body(t, carry):
        row = (sid + t) & (NS - 1)
        bv = bndall_v[pl.ds(row * 16, 16)]
        lo_t = jnp.min(bv)
        hi_t = jnp.max(bv)

        @pl.when((lo_t <= ss) & (ss < hi_t))
        def _add_row():
            k_row = (ss - lo_t) // STR

            @pl.when(k_row < WCAP)
            def _from_spmem():
                pltpu.sync_copy(
                    sstage.at[pl.ds(row * (WCAP * STR) + k_row * STR, STR)],
                    tmp_v)

            @pl.when(k_row >= WCAP)
            def _from_hbm():
                pltpu.sync_copy(
                    shared.at[pl.ds((row * NC + cid) * HROWS + ss, STR)],
                    tmp_v)

            for j in range(STR // (16 * ZU)):
                off = j * (16 * ZU)
                for u in range(ZU):
                    sl = pl.ds(ss + off + u * 16, 16)
                    tl = pl.ds(off + u * 16, 16)
                    hist_v[sl] = hist_v[sl] + tmp_v[tl]

        return carry

    lax.fori_loop(1, NS, red_body, 0)

    pltpu.sync_copy(hist_v.at[pl.ds(ss, STR)],
                    parts_hbm.at[pl.ds(cid * HROWS + ss, STR)])


def _dense_body(parts_ref, embed_ref, w1_ref, b1_ref, w2_ref, b2_ref,
                we_ref, be_ref, ep_ref, wh1_ref, bh1_ref, wh2_ref, bh2_ref,
                wh3_ref, bh3_ref, err_ref, qm_ref, gf_ref):
    ddot = functools.partial(jnp.dot, preferred_element_type=jnp.float32)
    counts = jnp.reshape(parts_ref[pl.ds(0, HROWS)]
                         + parts_ref[pl.ds(HROWS, HROWS)], (B, ROW))
    t = _silu(ddot(embed_ref[...], w1_ref[...]) + b1_ref[...])
    t = _silu(ddot(t, w2_ref[...]) + b2_ref[...])
    gf = ddot(counts, t)
    qm = ddot(gf, we_ref[...]) + be_ref[...]
    qe = jnp.concatenate([qm, ep_ref[...]], axis=1)
    x = (ddot(gf, wh1_ref[pl.ds(0, 128), :])
         + ddot(qe, wh1_ref[pl.ds(128, 2), :]) + bh1_ref[...])
    x = _silu(x)
    x = _silu(ddot(x, wh2_ref[...]) + bh2_ref[...])
    e = ddot(x, wh3_ref[...]) + bh3_ref[...]
    s = jnp.maximum(e, 0.0) + jnp.log(1.0 + jnp.exp(-jnp.abs(e)))
    eq = jnp.transpose(jnp.concatenate([s, qm], axis=1))
    err_ref[...] = eq[0]
    qm_ref[...] = eq[1]
    gf_ref[...] = gf


def kernel(atomic_numbers, batch, energy_pred, embed, W1, b1, W2, b2,
           We, be, Wh1, bh1, Wh2, bh2, Wh3, bh3):
    mesh = plsc.VectorSubcoreMesh(core_axis_name="c", subcore_axis_name="s",
                                  num_cores=NC, num_subcores=NS)
    parts = pl.kernel(
        _hist_body,
        out_type=jax.ShapeDtypeStruct((NC * HROWS,), jnp.float32),
        mesh=mesh,
        compiler_params=pltpu.CompilerParams(needs_layout_passes=False),
        scratch_types=[
            pltpu.VMEM((CHUNK,), jnp.int32),
            pltpu.VMEM((CHUNK,), jnp.int32),
            pltpu.VMEM((HSZ,), jnp.float32),
            pltpu.VMEM((STR,), jnp.float32),
            pltpu.VMEM((16,), jnp.int32),
            pltpu.VMEM((NS * 16,), jnp.int32),
            pltpu.HBM((NW * HROWS,), jnp.float32),
            pltpu.VMEM_SHARED((NS * WCAP * STR,), jnp.float32),
            pltpu.VMEM_SHARED((NS * 16,), jnp.int32),
            pltpu.SemaphoreType.DMA,
            pltpu.SemaphoreType.DMA,
        ],
    )(batch.astype(jnp.int32), atomic_numbers.astype(jnp.int32))

    embed_p = jnp.pad(embed, ((0, ROW - NZ), (0, 0)))
    err, qm, gf = pl.pallas_call(
        _dense_body,
        out_shape=[
            jax.ShapeDtypeStruct((B,), jnp.float32),
            jax.ShapeDtypeStruct((B,), jnp.float32),
            jax.ShapeDtypeStruct((B, 128), jnp.float32),
        ],
    )(parts, embed_p, W1, b1, W2, b2, We, be,
      energy_pred.reshape(B, 1), Wh1, bh1, Wh2, bh2, Wh3, bh3)

    return err, qm, gf
